# baseline (speedup 1.0000x reference)
import jax
import jax.numpy as jnp
from jax.experimental import pallas as pl
from jax.experimental.pallas import tpu as pltpu

B, T, C, H, W = 32, 256, 3, 32, 32
CHW = C * H * W
FEAT = 512
HID = 128
TOP_K = 8
G3 = 3 * HID


def _dot_nt(a, b):
    return jax.lax.dot_general(a, b, (((1,), (1,)), ((), ())),
                               preferred_element_type=jnp.float32)


TT = 8


def _enc_kernel(x_ref, wenc_ref, benc_ref, wih_ref, bih_ref, out_ref):
    x = x_ref[...].reshape(B * TT, CHW)
    f = _dot_nt(x, wenc_ref[...]) + benc_ref[...]
    g = _dot_nt(f, wih_ref[...]) + bih_ref[...]
    out_ref[...] = jnp.transpose(g.reshape(B, TT, 2 * G3), (1, 0, 2))


def _seq_kernel(gi_ref, whhf_ref, whhb_ref, bhhf_ref, bhhb_ref,
                lenc_ref, lenr_ref, temp_ref, b2_ref,
                w1_ref, b1_ref, w2r_ref, whead_ref, bhead_ref,
                out_ref, outs_ref):
    whhf = whhf_ref[...].astype(jnp.bfloat16)
    whhb = whhb_ref[...].astype(jnp.bfloat16)
    bhhf = bhhf_ref[...]
    bhhb = bhhb_ref[...]
    lenc = lenc_ref[...]

    def cell(gi_t, h, whh, bhh):
        gh = _dot_nt(h.astype(jnp.bfloat16), whh) + bhh
        r = jax.nn.sigmoid(gi_t[:, :HID] + gh[:, :HID])
        z = jax.nn.sigmoid(gi_t[:, HID:2 * HID] + gh[:, HID:2 * HID])
        n = jnp.tanh(gi_t[:, 2 * HID:] + r * gh[:, 2 * HID:])
        return (1.0 - z) * n + z * h

    def step(t, carry):
        hf, hb = carry
        tb = T - 1 - t
        gif = gi_ref[pl.ds(t * B, B), :G3]
        gib = gi_ref[pl.ds(tb * B, B), G3:]
        hfn = cell(gif, hf, whhf, bhhf)
        hbn = cell(gib, hb, whhb, bhhb)
        vf = t < lenc
        vb = tb < lenc
        outs_ref[pl.ds(t * B, B), :HID] = jnp.where(vf, hfn, 0.0)
        outs_ref[pl.ds(tb * B, B), HID:] = jnp.where(vb, hbn, 0.0)
        return jnp.where(vf, hfn, hf), jnp.where(vb, hbn, hb)

    h0 = jnp.zeros((B, HID), jnp.float32)
    jax.lax.fori_loop(0, T, step, (h0, h0))

    flat = outs_ref[...]
    h1 = jnp.maximum(_dot_nt(flat, w1_ref[...]) + b1_ref[...], 0.0)
    m = jnp.dot(h1, w2r_ref[...], preferred_element_type=jnp.float32)

    ridx = jax.lax.broadcasted_iota(jnp.int32, (T * B, B), 0)
    lidx = jax.lax.broadcasted_iota(jnp.int32, (T * B, B), 1)
    eyer = (ridx % B) == lidx
    msel = jnp.where(eyer, m, 0.0)
    scores = jnp.sum(msel.reshape(T, B, B), axis=1) + b2_ref[...]

    lenr = lenr_ref[...]
    tio = jax.lax.broadcasted_iota(jnp.int32, (T, B), 0)
    valid = tio < lenr
    temp = jnp.clip(temp_ref[...], 0.001, 10.0)
    logits = jnp.where(valid, scores, -jnp.inf) / temp
    mx = jnp.max(logits, axis=0, keepdims=True)
    e = jnp.exp(logits - mx)
    probs = e / jnp.sum(e, axis=0, keepdims=True)

    kact = jnp.minimum(lenr, TOP_K)
    work = probs
    acc = jnp.zeros((T, B), jnp.float32)
    vsum = jnp.zeros((1, B), jnp.float32)
    for i in range(TOP_K):
        v = jnp.max(work, axis=0, keepdims=True)
        hit = work == v
        idx = jnp.min(jnp.where(hit, tio, T), axis=0, keepdims=True)
        onehot = tio == idx
        ind = (i < kact).astype(jnp.float32)
        acc = acc + jnp.where(onehot, v * ind, 0.0)
        vsum = vsum + v * ind
        work = jnp.where(onehot, -1.0, work)
    att = acc / jnp.maximum(vsum, 1e-12)
    maskf = valid.astype(jnp.float32)
    uni = maskf / (jnp.sum(maskf, axis=0, keepdims=True) + 1e-8)
    att = jnp.where(vsum > 1e-8, att, uni)

    attr = jnp.broadcast_to(att[:, None, :], (T, B, B)).reshape(T * B, B)
    attc = jnp.sum(jnp.where(eyer, attr, 0.0), axis=1, keepdims=True)
    seq = jnp.sum((flat * attc).reshape(T, B, 2 * HID), axis=0)
    out_ref[...] = _dot_nt(seq, whead_ref[...]) + bhead_ref[...]


def kernel(frames, params, lengths):
    p = params
    x = frames.reshape(B, T, CHW)
    wih = jnp.concatenate([p['gru_fwd']['W_ih'], p['gru_bwd']['W_ih']], axis=0)
    bih = jnp.concatenate([p['gru_fwd']['b_ih'], p['gru_bwd']['b_ih']]).reshape(1, 2 * G3)
    benc = p['b_enc'].reshape(1, FEAT)

    gi_tb = pl.pallas_call(
        _enc_kernel,
        grid=(T // TT,),
        in_specs=[
            pl.BlockSpec((B, TT, CHW), lambda i: (0, i, 0)),
            pl.BlockSpec((FEAT, CHW), lambda i: (0, 0)),
            pl.BlockSpec((1, FEAT), lambda i: (0, 0)),
            pl.BlockSpec((2 * G3, FEAT), lambda i: (0, 0)),
            pl.BlockSpec((1, 2 * G3), lambda i: (0, 0)),
        ],
        out_specs=pl.BlockSpec((TT, B, 2 * G3), lambda i: (i, 0, 0)),
        out_shape=jax.ShapeDtypeStruct((T, B, 2 * G3), jnp.float32),
    )(x, p['W_enc'], benc, wih, bih)

    gi_flat = gi_tb.reshape(T * B, 2 * G3)

    bhhf = p['gru_fwd']['b_hh'].reshape(1, G3)
    bhhb = p['gru_bwd']['b_hh'].reshape(1, G3)
    lenc = lengths.reshape(B, 1)
    lenr = lengths.reshape(1, B)
    tempc = p['temperature'].reshape(1, 1)
    b1 = p['b1'].reshape(1, 64)
    w2r = jnp.broadcast_to(p['W2'].reshape(64, 1), (64, B))
    b2 = p['b2'].reshape(1, 1)
    whead = jnp.concatenate([p['Wt'], p['Wo']], axis=0)
    bhead = jnp.concatenate([p['bt'], p['bo']]).reshape(1, 21)

    heads = pl.pallas_call(
        _seq_kernel,
        out_shape=jax.ShapeDtypeStruct((B, 21), jnp.float32),
        scratch_shapes=[pltpu.VMEM((T * B, 2 * HID), jnp.float32)],
    )(gi_flat, p['gru_fwd']['W_hh'], p['gru_bwd']['W_hh'], bhhf, bhhb,
      lenc, lenr, tempc, b2, p['W1'], b1, w2r, whead, bhead)

    return heads[:, :11], heads[:, 11:21]


# speedup vs baseline: 1.0724x; 1.0724x over previous
import jax
import jax.numpy as jnp
from jax.experimental import pallas as pl
from jax.experimental.pallas import tpu as pltpu

B, T, C, H, W = 32, 256, 3, 32, 32
CHW = C * H * W
FEAT = 512
HID = 128
TOP_K = 8
G3 = 3 * HID


def _dot_nt(a, b):
    return jax.lax.dot_general(a, b, (((1,), (1,)), ((), ())),
                               preferred_element_type=jnp.float32)


TT = 8


def _enc_kernel(x_ref, wenc_ref, benc_ref, wih_ref, bih_ref, out_ref):
    x = x_ref[...].reshape(B * TT, CHW)
    f = _dot_nt(x, wenc_ref[...]) + benc_ref[...]
    g = _dot_nt(f, wih_ref[...]) + bih_ref[...]
    out_ref[...] = jnp.transpose(g.reshape(B, TT, 2 * G3), (1, 0, 2))


def _seq_kernel(gi_ref, whhf_ref, whhb_ref, bhhf_ref, bhhb_ref,
                lenc_ref, lenr_ref, temp_ref, b2_ref,
                w1_ref, b1_ref, w2r_ref, whead_ref, bhead_ref,
                out_ref, outs_ref):
    whhf = whhf_ref[...]
    whhb = whhb_ref[...]
    bhhf = bhhf_ref[...]
    bhhb = bhhb_ref[...]
    lenc = lenc_ref[...]

    def cell(gi_t, h, whh, bhh):
        gh = jnp.dot(h.astype(jnp.bfloat16), whh,
                     preferred_element_type=jnp.float32) + bhh
        r = jax.nn.sigmoid(gi_t[:, :HID] + gh[:, :HID])
        z = jax.nn.sigmoid(gi_t[:, HID:2 * HID] + gh[:, HID:2 * HID])
        n = jnp.tanh(gi_t[:, 2 * HID:] + r * gh[:, 2 * HID:])
        return (1.0 - z) * n + z * h

    def substep(t, hf, hb):
        tb = T - 1 - t
        gif = gi_ref[pl.ds(t * B, B), :G3]
        gib = gi_ref[pl.ds(tb * B, B), G3:]
        hfn = cell(gif, hf, whhf, bhhf)
        hbn = cell(gib, hb, whhb, bhhb)
        vf = t < lenc
        vb = tb < lenc
        outs_ref[pl.ds(t * B, B), :HID] = jnp.where(vf, hfn, 0.0)
        outs_ref[pl.ds(tb * B, B), HID:] = jnp.where(vb, hbn, 0.0)
        return jnp.where(vf, hfn, hf), jnp.where(vb, hbn, hb)

    def step(i, carry):
        hf, hb = carry
        for j in range(8):
            hf, hb = substep(8 * i + j, hf, hb)
        return hf, hb

    h0 = jnp.zeros((B, HID), jnp.float32)
    jax.lax.fori_loop(0, T // 8, step, (h0, h0))

    flat = outs_ref[...]
    h1 = jnp.maximum(_dot_nt(flat, w1_ref[...]) + b1_ref[...], 0.0)
    m = jnp.dot(h1, w2r_ref[...], preferred_element_type=jnp.float32)

    ridx = jax.lax.broadcasted_iota(jnp.int32, (T * B, B), 0)
    lidx = jax.lax.broadcasted_iota(jnp.int32, (T * B, B), 1)
    eyer = (ridx % B) == lidx
    msel = jnp.where(eyer, m, 0.0)
    scores = jnp.sum(msel.reshape(T, B, B), axis=1) + b2_ref[...]

    lenr = lenr_ref[...]
    tio = jax.lax.broadcasted_iota(jnp.int32, (T, B), 0)
    valid = tio < lenr
    temp = jnp.clip(temp_ref[...], 0.001, 10.0)
    logits = jnp.where(valid, scores, -jnp.inf) / temp
    mx = jnp.max(logits, axis=0, keepdims=True)
    e = jnp.exp(logits - mx)
    probs = e / jnp.sum(e, axis=0, keepdims=True)

    kact = jnp.minimum(lenr, TOP_K)
    work = probs
    acc = jnp.zeros((T, B), jnp.float32)
    vsum = jnp.zeros((1, B), jnp.float32)
    for i in range(TOP_K):
        v = jnp.max(work, axis=0, keepdims=True)
        hit = work == v
        idx = jnp.min(jnp.where(hit, tio, T), axis=0, keepdims=True)
        onehot = tio == idx
        ind = (i < kact).astype(jnp.float32)
        acc = acc + jnp.where(onehot, v * ind, 0.0)
        vsum = vsum + v * ind
        work = jnp.where(onehot, -1.0, work)
    att = acc / jnp.maximum(vsum, 1e-12)
    maskf = valid.astype(jnp.float32)
    uni = maskf / (jnp.sum(maskf, axis=0, keepdims=True) + 1e-8)
    att = jnp.where(vsum > 1e-8, att, uni)

    attr = jnp.broadcast_to(att[:, None, :], (T, B, B)).reshape(T * B, B)
    attc = jnp.sum(jnp.where(eyer, attr, 0.0), axis=1, keepdims=True)
    seq = jnp.sum((flat * attc).reshape(T, B, 2 * HID), axis=0)
    out_ref[...] = _dot_nt(seq, whead_ref[...]) + bhead_ref[...]


def kernel(frames, params, lengths):
    p = params
    x = frames.reshape(B, T, CHW)
    wih = jnp.concatenate([p['gru_fwd']['W_ih'], p['gru_bwd']['W_ih']], axis=0)
    bih = jnp.concatenate([p['gru_fwd']['b_ih'], p['gru_bwd']['b_ih']]).reshape(1, 2 * G3)
    benc = p['b_enc'].reshape(1, FEAT)

    gi_tb = pl.pallas_call(
        _enc_kernel,
        grid=(T // TT,),
        in_specs=[
            pl.BlockSpec((B, TT, CHW), lambda i: (0, i, 0)),
            pl.BlockSpec((FEAT, CHW), lambda i: (0, 0)),
            pl.BlockSpec((1, FEAT), lambda i: (0, 0)),
            pl.BlockSpec((2 * G3, FEAT), lambda i: (0, 0)),
            pl.BlockSpec((1, 2 * G3), lambda i: (0, 0)),
        ],
        out_specs=pl.BlockSpec((TT, B, 2 * G3), lambda i: (i, 0, 0)),
        out_shape=jax.ShapeDtypeStruct((T, B, 2 * G3), jnp.float32),
    )(x, p['W_enc'], benc, wih, bih)

    gi_flat = gi_tb.reshape(T * B, 2 * G3)

    bhhf = p['gru_fwd']['b_hh'].reshape(1, G3)
    bhhb = p['gru_bwd']['b_hh'].reshape(1, G3)
    lenc = lengths.reshape(B, 1)
    lenr = lengths.reshape(1, B)
    tempc = p['temperature'].reshape(1, 1)
    b1 = p['b1'].reshape(1, 64)
    w2r = jnp.broadcast_to(p['W2'].reshape(64, 1), (64, B))
    b2 = p['b2'].reshape(1, 1)
    whead = jnp.concatenate([p['Wt'], p['Wo']], axis=0)
    bhead = jnp.concatenate([p['bt'], p['bo']]).reshape(1, 21)

    heads = pl.pallas_call(
        _seq_kernel,
        out_shape=jax.ShapeDtypeStruct((B, 21), jnp.float32),
        scratch_shapes=[pltpu.VMEM((T * B, 2 * HID), jnp.float32)],
    )(gi_flat, p['gru_fwd']['W_hh'].T.astype(jnp.bfloat16),
      p['gru_bwd']['W_hh'].T.astype(jnp.bfloat16), bhhf, bhhb,
      lenc, lenr, tempc, b2, p['W1'], b1, w2r, whead, bhead)

    return heads[:, :11], heads[:, 11:21]


# speedup vs baseline: 1.1495x; 1.0719x over previous
import jax
import jax.numpy as jnp
from jax.experimental import pallas as pl
from jax.experimental.pallas import tpu as pltpu

B, T, C, H, W = 32, 256, 3, 32, 32
CHW = C * H * W
FEAT = 512
HID = 128
TOP_K = 8
G3 = 3 * HID


def _dot_nt(a, b):
    return jax.lax.dot_general(a, b, (((1,), (1,)), ((), ())),
                               preferred_element_type=jnp.float32)


TT = 32


def _enc_kernel(x_ref, wenc_ref, benc_ref, wih_ref, bih_ref, out_ref):
    x = x_ref[...].reshape(B * TT, CHW)
    f = _dot_nt(x, wenc_ref[...]) + benc_ref[...]
    g = _dot_nt(f, wih_ref[...]) + bih_ref[...]
    out_ref[...] = jnp.transpose(g.reshape(B, TT, 2 * G3), (1, 0, 2))


def _seq_kernel(gi_ref, whhf_ref, whhb_ref, bhhf_ref, bhhb_ref,
                lenc_ref, lenr_ref, temp_ref, b2_ref,
                w1_ref, b1_ref, w2r_ref, whead_ref, bhead_ref,
                out_ref, outs_ref):
    whhf = whhf_ref[...]
    whhb = whhb_ref[...]
    bhhf = bhhf_ref[...]
    bhhb = bhhb_ref[...]
    lenc = lenc_ref[...]

    def cell(gi_t, h, whh, bhh):
        gh = jnp.dot(h.astype(jnp.bfloat16), whh,
                     preferred_element_type=jnp.float32) + bhh
        r = jax.nn.sigmoid(gi_t[:, :HID] + gh[:, :HID])
        z = jax.nn.sigmoid(gi_t[:, HID:2 * HID] + gh[:, HID:2 * HID])
        n = jnp.tanh(gi_t[:, 2 * HID:] + r * gh[:, 2 * HID:])
        return (1.0 - z) * n + z * h

    def substep(t, hf, hb):
        tb = T - 1 - t
        gif = gi_ref[pl.ds(t * B, B), :G3]
        gib = gi_ref[pl.ds(tb * B, B), G3:]
        hfn = cell(gif, hf, whhf, bhhf)
        hbn = cell(gib, hb, whhb, bhhb)
        vf = t < lenc
        vb = tb < lenc
        outs_ref[pl.ds(t * B, B), :HID] = jnp.where(vf, hfn, 0.0)
        outs_ref[pl.ds(tb * B, B), HID:] = jnp.where(vb, hbn, 0.0)
        return jnp.where(vf, hfn, hf), jnp.where(vb, hbn, hb)

    def step(i, carry):
        hf, hb = carry
        for j in range(8):
            hf, hb = substep(8 * i + j, hf, hb)
        return hf, hb

    h0 = jnp.zeros((B, HID), jnp.float32)
    jax.lax.fori_loop(0, T // 8, step, (h0, h0))

    flat = outs_ref[...]
    h1 = jnp.maximum(_dot_nt(flat, w1_ref[...]) + b1_ref[...], 0.0)
    m = jnp.dot(h1, w2r_ref[...], preferred_element_type=jnp.float32)

    ridx = jax.lax.broadcasted_iota(jnp.int32, (T * B, B), 0)
    lidx = jax.lax.broadcasted_iota(jnp.int32, (T * B, B), 1)
    eyer = (ridx % B) == lidx
    msel = jnp.where(eyer, m, 0.0)
    scores = jnp.sum(msel.reshape(T, B, B), axis=1) + b2_ref[...]

    lenr = lenr_ref[...]
    tio = jax.lax.broadcasted_iota(jnp.int32, (T, B), 0)
    valid = tio < lenr
    temp = jnp.clip(temp_ref[...], 0.001, 10.0)
    logits = jnp.where(valid, scores, -jnp.inf) / temp
    mx = jnp.max(logits, axis=0, keepdims=True)
    e = jnp.exp(logits - mx)
    probs = e / jnp.sum(e, axis=0, keepdims=True)

    kact = jnp.minimum(lenr, TOP_K)
    work = probs
    acc = jnp.zeros((T, B), jnp.float32)
    vsum = jnp.zeros((1, B), jnp.float32)
    for i in range(TOP_K):
        v = jnp.max(work, axis=0, keepdims=True)
        hit = work == v
        idx = jnp.min(jnp.where(hit, tio, T), axis=0, keepdims=True)
        onehot = tio == idx
        ind = (i < kact).astype(jnp.float32)
        acc = acc + jnp.where(onehot, v * ind, 0.0)
        vsum = vsum + v * ind
        work = jnp.where(onehot, -1.0, work)
    att = acc / jnp.maximum(vsum, 1e-12)
    maskf = valid.astype(jnp.float32)
    uni = maskf / (jnp.sum(maskf, axis=0, keepdims=True) + 1e-8)
    att = jnp.where(vsum > 1e-8, att, uni)

    attr = jnp.broadcast_to(att[:, None, :], (T, B, B)).reshape(T * B, B)
    attc = jnp.sum(jnp.where(eyer, attr, 0.0), axis=1, keepdims=True)
    seq = jnp.sum((flat * attc).reshape(T, B, 2 * HID), axis=0)
    out_ref[...] = _dot_nt(seq, whead_ref[...]) + bhead_ref[...]


def kernel(frames, params, lengths):
    p = params
    x = frames.reshape(B, T, CHW)
    wih = jnp.concatenate([p['gru_fwd']['W_ih'], p['gru_bwd']['W_ih']], axis=0)
    bih = jnp.concatenate([p['gru_fwd']['b_ih'], p['gru_bwd']['b_ih']]).reshape(1, 2 * G3)
    benc = p['b_enc'].reshape(1, FEAT)

    gi_tb = pl.pallas_call(
        _enc_kernel,
        grid=(T // TT,),
        in_specs=[
            pl.BlockSpec((B, TT, CHW), lambda i: (0, i, 0)),
            pl.BlockSpec((FEAT, CHW), lambda i: (0, 0)),
            pl.BlockSpec((1, FEAT), lambda i: (0, 0)),
            pl.BlockSpec((2 * G3, FEAT), lambda i: (0, 0)),
            pl.BlockSpec((1, 2 * G3), lambda i: (0, 0)),
        ],
        out_specs=pl.BlockSpec((TT, B, 2 * G3), lambda i: (i, 0, 0)),
        out_shape=jax.ShapeDtypeStruct((T, B, 2 * G3), jnp.float32),
    )(x, p['W_enc'], benc, wih, bih)

    gi_flat = gi_tb.reshape(T * B, 2 * G3)

    bhhf = p['gru_fwd']['b_hh'].reshape(1, G3)
    bhhb = p['gru_bwd']['b_hh'].reshape(1, G3)
    lenc = lengths.reshape(B, 1)
    lenr = lengths.reshape(1, B)
    tempc = p['temperature'].reshape(1, 1)
    b1 = p['b1'].reshape(1, 64)
    w2r = jnp.broadcast_to(p['W2'].reshape(64, 1), (64, B))
    b2 = p['b2'].reshape(1, 1)
    whead = jnp.concatenate([p['Wt'], p['Wo']], axis=0)
    bhead = jnp.concatenate([p['bt'], p['bo']]).reshape(1, 21)

    heads = pl.pallas_call(
        _seq_kernel,
        out_shape=jax.ShapeDtypeStruct((B, 21), jnp.float32),
        scratch_shapes=[pltpu.VMEM((T * B, 2 * HID), jnp.float32)],
    )(gi_flat, p['gru_fwd']['W_hh'].T.astype(jnp.bfloat16),
      p['gru_bwd']['W_hh'].T.astype(jnp.bfloat16), bhhf, bhhb,
      lenc, lenr, tempc, b2, p['W1'], b1, w2r, whead, bhead)

    return heads[:, :11], heads[:, 11:21]
